# baseline (speedup 1.0000x reference)
import functools

import jax
import jax.numpy as jnp
from jax import lax
from jax.experimental import pallas as pl
from jax.experimental.pallas import tpu as pltpu
from jax.experimental.pallas import tpu_sc as plsc

_CHUNK = 16
_NBUF = 4


def _embedding_lookup(tokens_flat, table):
    B, = tokens_flat.shape
    V, D = table.shape
    info = plsc.get_sparse_core_info()
    NC, NS = info.num_cores, info.num_subcores
    NW = NC * NS
    b_per_w = B // NW
    n_chunks = b_per_w // _CHUNK
    assert B == NW * b_per_w and b_per_w == n_chunks * _CHUNK
    assert n_chunks % _NBUF == 0 and n_chunks >= 2 * _NBUF

    mesh = plsc.VectorSubcoreMesh(core_axis_name="c", subcore_axis_name="s")

    @functools.partial(
        pl.kernel,
        mesh=mesh,
        out_type=jax.ShapeDtypeStruct((B, D), jnp.float32),
        scratch_types=[
            pltpu.VMEM((b_per_w,), jnp.int32),
        ]
        + [pltpu.VMEM((_CHUNK, D), jnp.float32)] * _NBUF
        + [pltpu.VMEM_SHARED((NS, 2, _CHUNK, D), jnp.float32)]
        + [pltpu.SemaphoreType.DMA] * (_NBUF + 6),
    )
    def gather_kernel(idx_hbm, table_hbm, out_hbm, idx_v, *bufs_sems):
        bufs = bufs_sems[:_NBUF]
        shared = bufs_sems[_NBUF]
        gsems = bufs_sems[_NBUF + 1:2 * _NBUF + 1]
        wsems = bufs_sems[2 * _NBUF + 1:2 * _NBUF + 3]
        ssems = bufs_sems[2 * _NBUF + 3:2 * _NBUF + 5]
        dsems = bufs_sems[2 * _NBUF + 5:2 * _NBUF + 7]
        sid = lax.axis_index("s")
        wid = sid * NC + lax.axis_index("c")
        base = wid * b_per_w
        pltpu.sync_copy(idx_hbm.at[pl.ds(base, b_per_w)], idx_v)

        def out_slice(i):
            return out_hbm.at[pl.ds(base + i * _CHUNK, _CHUNK)]

        def slot(s):
            return shared.at[sid, s]

        def start_gather(i, b):
            off = pl.multiple_of(i * _CHUNK, _CHUNK)
            pltpu.async_copy(table_hbm.at[idx_v.at[pl.ds(off, _CHUNK)]],
                             bufs[b], gsems[b])

        for b in range(_NBUF - 1):
            start_gather(b, b)

        def step(i, b, first=False, last=False, has_prev=True):
            pltpu.make_async_copy(table_hbm.at[idx_v.at[pl.ds(0, _CHUNK)]],
                                  bufs[b], gsems[b]).wait()
            if not last:
                nb = (b + _NBUF - 1) % _NBUF
                if nb % 2 == 0 and has_prev:
                    pltpu.make_async_copy(bufs[nb], out_slice(0),
                                          wsems[nb // 2]).wait()
                start_gather(i + _NBUF - 1, nb)
            if b % 2 == 0:
                pltpu.async_copy(bufs[b], out_slice(i), wsems[b // 2])
            else:
                s = (b - 1) // 2
                if not first:
                    pltpu.make_async_copy(slot(s), out_slice(0),
                                          dsems[s]).wait()
                pltpu.async_copy(bufs[b], slot(s), ssems[s])
                pltpu.make_async_copy(bufs[b], slot(s), ssems[s]).wait()
                pltpu.async_copy(slot(s), out_slice(i), dsems[s])

        for i in range(_NBUF):
            step(i, i, first=True, has_prev=(i > 0))

        n_steady = (n_chunks - 2 * _NBUF) // _NBUF

        def body(grp, carry):
            for k in range(_NBUF):
                step(_NBUF + _NBUF * grp + k, k)
            return carry

        lax.fori_loop(0, n_steady, body, 0)

        for i in range(n_chunks - _NBUF, n_chunks):
            step(i, i % _NBUF, last=(i + _NBUF - 1 >= n_chunks))

        for s in range(2):
            pltpu.make_async_copy(bufs[2 * s], out_slice(0), wsems[s]).wait()
            pltpu.make_async_copy(slot(s), out_slice(0), dsems[s]).wait()

    return gather_kernel(tokens_flat, table)


def kernel(tokens, start_pos, tok_embeddings_weight):
    B, S = tokens.shape
    V, D = tok_embeddings_weight.shape
    out = _embedding_lookup(tokens.reshape(B * S), tok_embeddings_weight)
    return out.reshape(B, S, D)


# speedup vs baseline: 1.0106x; 1.0106x over previous
import functools

import jax
import jax.numpy as jnp
from jax import lax
from jax.experimental import pallas as pl
from jax.experimental.pallas import tpu as pltpu
from jax.experimental.pallas import tpu_sc as plsc

_CHUNK = 16
_NBUF = 4
_NSLOT = 2


def _embedding_lookup(tokens_flat, table):
    B, = tokens_flat.shape
    V, D = table.shape
    info = plsc.get_sparse_core_info()
    NC, NS = info.num_cores, info.num_subcores
    NW = NC * NS
    b_per_w = B // NW
    n_chunks = b_per_w // _CHUNK
    assert B == NW * b_per_w and b_per_w == n_chunks * _CHUNK
    assert n_chunks % _NBUF == 0 and _NBUF >= _NSLOT

    mesh = plsc.VectorSubcoreMesh(core_axis_name="c", subcore_axis_name="s")

    @functools.partial(
        pl.kernel,
        mesh=mesh,
        out_type=jax.ShapeDtypeStruct((B, D), jnp.float32),
        scratch_types=[
            pltpu.VMEM((b_per_w,), jnp.int32),
        ]
        + [pltpu.VMEM((_CHUNK, D), jnp.float32)] * _NBUF
        + [pltpu.VMEM_SHARED((NS, _NSLOT, _CHUNK, D), jnp.float32)]
        + [pltpu.SemaphoreType.DMA] * (2 * _NBUF + _NSLOT),
    )
    def gather_kernel(idx_hbm, table_hbm, out_hbm, idx_v, *bufs_sems):
        bufs = bufs_sems[:_NBUF]
        shared = bufs_sems[_NBUF]
        gsems = bufs_sems[_NBUF + 1:2 * _NBUF + 1]
        ssems = bufs_sems[2 * _NBUF + 1:3 * _NBUF + 1]
        dsems = bufs_sems[3 * _NBUF + 1:]
        sid = lax.axis_index("s")
        wid = sid * NC + lax.axis_index("c")
        base = wid * b_per_w
        pltpu.sync_copy(idx_hbm.at[pl.ds(base, b_per_w)], idx_v)

        def out_slice(i):
            return out_hbm.at[pl.ds(base + i * _CHUNK, _CHUNK)]

        def slot(s):
            return shared.at[sid, s]

        def start_gather(i, b):
            off = pl.multiple_of(i * _CHUNK, _CHUNK)
            pltpu.async_copy(table_hbm.at[idx_v.at[pl.ds(off, _CHUNK)]],
                             bufs[b], gsems[b])

        for b in range(_NBUF - 1):
            start_gather(b, b)

        def step(i, b, s, first=False, last=False):
            pltpu.make_async_copy(table_hbm.at[idx_v.at[pl.ds(0, _CHUNK)]],
                                  bufs[b], gsems[b]).wait()
            if not last:
                start_gather(i + _NBUF - 1, (b + _NBUF - 1) % _NBUF)
            if not first:
                pltpu.make_async_copy(slot(s), out_slice(0), dsems[s]).wait()
            pltpu.async_copy(bufs[b], slot(s), ssems[b])
            pltpu.make_async_copy(bufs[b], slot(s), ssems[b]).wait()
            pltpu.async_copy(slot(s), out_slice(i), dsems[s])

        for i in range(_NSLOT):
            step(i, i % _NBUF, i % _NSLOT, first=True)

        n_steady = (n_chunks - 2 * _NBUF) // _NBUF

        def body(grp, carry):
            for k in range(_NBUF):
                i = _NSLOT + _NBUF * grp + k
                step(i, (_NSLOT + k) % _NBUF, (_NSLOT + k) % _NSLOT)
            return carry

        lax.fori_loop(0, n_steady, body, 0)

        for i in range(_NSLOT + n_steady * _NBUF, n_chunks):
            step(i, i % _NBUF, i % _NSLOT, last=(i + _NBUF - 1 >= n_chunks))

        for s in range(_NSLOT):
            pltpu.make_async_copy(slot(s), out_slice(0), dsems[s]).wait()

    return gather_kernel(tokens_flat, table)


def kernel(tokens, start_pos, tok_embeddings_weight):
    B, S = tokens.shape
    V, D = tok_embeddings_weight.shape
    out = _embedding_lookup(tokens.reshape(B * S), tok_embeddings_weight)
    return out.reshape(B, S, D)
